# baseline (speedup 1.0000x reference)
import functools

import jax
import jax.numpy as jnp
from jax import lax
from jax.experimental import pallas as pl
from jax.experimental.pallas import tpu as pltpu
from jax.experimental.pallas import tpu_sc as plsc

_NC, _NS, _L = 2, 16, 16
_NW = _NC * _NS
_CH = 20000
_VPI = 5


def _sc_body(x_hbm, t_hbm, sums_hbm, tv_hbm, buf0, buf1, out_v, tgt_v,
             idx_v, tvals_v, sem0, sem1, gsem, *, nrows, ncols):
    rw = nrows // _NW
    nch = ncols // _CH
    wid = lax.axis_index("s") * _NC + lax.axis_index("c")
    bufs = (buf0, buf1)
    sems = (sem0, sem1)
    base = wid * rw * ncols

    nchunks = rw * nch
    handles = [None, None]
    handles[0] = pltpu.async_copy(x_hbm.at[pl.ds(base, _CH)], bufs[0], sems[0])
    acc = None
    for k in range(nchunks):
        if k + 1 < nchunks:
            nb = (k + 1) % 2
            handles[nb] = pltpu.async_copy(
                x_hbm.at[pl.ds(base + (k + 1) * _CH, _CH)], bufs[nb], sems[nb]
            )
        cb = k % 2
        handles[cb].wait()
        buf = bufs[cb]
        if k % nch == 0:
            acc = tuple(jnp.zeros((_L,), jnp.float32) for _ in range(_VPI))

        def _step(i, a, _buf=buf):
            outs = []
            for v in range(_VPI):
                off = pl.multiple_of(i * (_L * _VPI) + v * _L, _L)
                outs.append(a[v] + jnp.exp(_buf[pl.ds(off, _L)]))
            return tuple(outs)

        acc = lax.fori_loop(0, _CH // (_L * _VPI), _step, acc)
        if (k + 1) % nch == 0:
            row = k // nch
            total = acc[0]
            for v in range(1, _VPI):
                total = total + acc[v]
            out_v[pl.ds(row * _L, _L)] = total
    pltpu.sync_copy(out_v, sums_hbm.at[pl.ds(wid * rw * _L, rw * _L)])

    @pl.when(wid == 0)
    def _gather_targets():
        pltpu.sync_copy(t_hbm, tgt_v)
        for i in range(nrows // _L):
            t16 = tgt_v[pl.ds(i * _L, _L)]
            row16 = lax.iota(jnp.int32, _L) + (i * _L)
            idx_v[pl.ds(i * _L, _L)] = row16 * ncols + t16
        pltpu.async_copy(x_hbm.at[idx_v], tvals_v, gsem).wait()
        pltpu.sync_copy(tvals_v, tv_hbm)


def _finish_kernel(s_ref, t_ref, o_ref):
    s = jnp.sum(s_ref[...], axis=1, keepdims=True)
    p = jnp.exp(t_ref[...]) / s
    o_ref[...] = -jnp.mean(jnp.log(p + 1e-6)).reshape(1, 1)


def kernel(input, target):
    b, c = input.shape
    x_flat = input.reshape(-1)
    tgt = target.astype(jnp.int32)
    mesh = plsc.VectorSubcoreMesh(core_axis_name="c", subcore_axis_name="s")
    rw = b // _NW

    sc = pl.kernel(
        functools.partial(_sc_body, nrows=b, ncols=c),
        out_type=[
            jax.ShapeDtypeStruct((b * _L,), jnp.float32),
            jax.ShapeDtypeStruct((b,), jnp.float32),
        ],
        mesh=mesh,
        scratch_types=[
            pltpu.VMEM((_CH,), jnp.float32),
            pltpu.VMEM((_CH,), jnp.float32),
            pltpu.VMEM((rw * _L,), jnp.float32),
            pltpu.VMEM((b,), jnp.int32),
            pltpu.VMEM((b,), jnp.int32),
            pltpu.VMEM((b,), jnp.float32),
            pltpu.SemaphoreType.DMA,
            pltpu.SemaphoreType.DMA,
            pltpu.SemaphoreType.DMA,
        ],
    )
    sums_flat, tvals = sc(x_flat, tgt)

    out = pl.pallas_call(
        _finish_kernel,
        out_shape=jax.ShapeDtypeStruct((1, 1), jnp.float32),
    )(sums_flat.reshape(b, _L), tvals.reshape(b, 1))
    return out[0, 0]


# speedup vs baseline: 1.0315x; 1.0315x over previous
import functools

import jax
import jax.numpy as jnp
from jax import lax
from jax.experimental import pallas as pl
from jax.experimental.pallas import tpu as pltpu
from jax.experimental.pallas import tpu_sc as plsc

_NC, _NS, _L = 2, 16, 16
_NW = _NC * _NS
_SPLIT = 72000
_SC_CH = 14000
_VPI = 5
_W = 4096
_K = 4


def _sc_body(x_hbm, t_hbm, sums_hbm, tv_hbm, buf0, buf1, out_v, tgt_v,
             idx_v, tvals_v, sem0, sem1, gsem, *, nrows, ncols, col0):
    rw = nrows // _NW
    seg = ncols - col0
    nch = seg // _SC_CH
    wid = lax.axis_index("s") * _NC + lax.axis_index("c")
    bufs = (buf0, buf1)
    sems = (sem0, sem1)
    row0 = wid * rw

    def _chunk_start(k):
        row = k // nch
        return (row0 + row) * ncols + col0 + (k % nch) * _SC_CH

    nchunks = rw * nch
    handles = [None, None]
    handles[0] = pltpu.async_copy(
        x_hbm.at[pl.ds(_chunk_start(0), _SC_CH)], bufs[0], sems[0]
    )
    acc = None
    for k in range(nchunks):
        if k + 1 < nchunks:
            nb = (k + 1) % 2
            handles[nb] = pltpu.async_copy(
                x_hbm.at[pl.ds(_chunk_start(k + 1), _SC_CH)], bufs[nb], sems[nb]
            )
        cb = k % 2
        handles[cb].wait()
        buf = bufs[cb]
        if k % nch == 0:
            acc = tuple(jnp.zeros((_L,), jnp.float32) for _ in range(_VPI))

        def _step(i, a, _buf=buf):
            outs = []
            for v in range(_VPI):
                off = pl.multiple_of(i * (_L * _VPI) + v * _L, _L)
                outs.append(a[v] + jnp.exp(_buf[pl.ds(off, _L)]))
            return tuple(outs)

        acc = lax.fori_loop(0, _SC_CH // (_L * _VPI), _step, acc)
        if (k + 1) % nch == 0:
            row = k // nch
            total = acc[0]
            for v in range(1, _VPI):
                total = total + acc[v]
            out_v[pl.ds(row * _L, _L)] = total
    pltpu.sync_copy(out_v, sums_hbm.at[pl.ds(row0 * _L, rw * _L)])

    @pl.when(wid == 0)
    def _gather_targets():
        pltpu.sync_copy(t_hbm, tgt_v)
        for i in range(nrows // _L):
            t16 = tgt_v[pl.ds(i * _L, _L)]
            row16 = lax.iota(jnp.int32, _L) + (i * _L)
            idx_v[pl.ds(i * _L, _L)] = row16 * ncols + t16
        pltpu.async_copy(x_hbm.at[idx_v], tvals_v, gsem).wait()
        pltpu.sync_copy(tvals_v, tv_hbm)


def _tc_sum_kernel(*refs, nsteps, width, ncols, nstreams):
    x_refs = refs[:nstreams]
    o_ref = refs[nstreams]
    acc = refs[nstreams + 1]
    j = pl.program_id(0)

    s = None
    for k in range(nstreams):
        x = x_refs[k][...]
        b, w = x.shape
        col = (j * nstreams + k) * width + jax.lax.broadcasted_iota(
            jnp.int32, (b, w), 1
        )
        e = jnp.where(col < ncols, jnp.exp(x), 0.0)
        sk = jnp.sum(e, axis=1, keepdims=True)
        s = sk if s is None else s + sk

    @pl.when(j == 0)
    def _init():
        acc[...] = s

    @pl.when(j > 0)
    def _accum():
        acc[...] += s

    @pl.when(j == nsteps - 1)
    def _finish():
        o_ref[...] = acc[...]


def _finish_kernel(sc_ref, tc_ref, t_ref, o_ref):
    s = jnp.sum(sc_ref[...], axis=1, keepdims=True) + tc_ref[...]
    p = jnp.exp(t_ref[...]) / s
    o_ref[...] = -jnp.mean(jnp.log(p + 1e-6)).reshape(1, 1)


def kernel(input, target):
    b, c = input.shape
    x_flat = input.reshape(-1)
    tgt = target.astype(jnp.int32)
    mesh = plsc.VectorSubcoreMesh(core_axis_name="c", subcore_axis_name="s")
    rw = b // _NW

    sc = pl.kernel(
        functools.partial(_sc_body, nrows=b, ncols=c, col0=_SPLIT),
        out_type=[
            jax.ShapeDtypeStruct((b * _L,), jnp.float32),
            jax.ShapeDtypeStruct((b,), jnp.float32),
        ],
        mesh=mesh,
        scratch_types=[
            pltpu.VMEM((_SC_CH,), jnp.float32),
            pltpu.VMEM((_SC_CH,), jnp.float32),
            pltpu.VMEM((rw * _L,), jnp.float32),
            pltpu.VMEM((b,), jnp.int32),
            pltpu.VMEM((b,), jnp.int32),
            pltpu.VMEM((b,), jnp.float32),
            pltpu.SemaphoreType.DMA,
            pltpu.SemaphoreType.DMA,
            pltpu.SemaphoreType.DMA,
        ],
    )
    sc_sums_flat, tvals = sc(x_flat, tgt)

    nsteps = pl.cdiv(_SPLIT, _W * _K)
    nblocks = pl.cdiv(_SPLIT, _W)

    def _x_spec(k):
        return pl.BlockSpec(
            (b, _W), lambda j, _k=k: (0, jnp.minimum(j * _K + _k, nblocks - 1))
        )

    tc_sums = pl.pallas_call(
        functools.partial(
            _tc_sum_kernel, nsteps=nsteps, width=_W, ncols=_SPLIT, nstreams=_K
        ),
        grid=(nsteps,),
        in_specs=[_x_spec(k) for k in range(_K)],
        out_specs=pl.BlockSpec((b, 1), lambda j: (0, 0)),
        out_shape=jax.ShapeDtypeStruct((b, 1), jnp.float32),
        scratch_shapes=[pltpu.VMEM((b, 1), jnp.float32)],
    )(*([input] * _K))

    out = pl.pallas_call(
        _finish_kernel,
        out_shape=jax.ShapeDtypeStruct((1, 1), jnp.float32),
    )(sc_sums_flat.reshape(b, _L), tc_sums, tvals.reshape(b, 1))
    return out[0, 0]


# speedup vs baseline: 2.3125x; 2.2418x over previous
import functools

import jax
import jax.numpy as jnp
from jax.experimental import pallas as pl
from jax.experimental.pallas import tpu as pltpu

_W = 4096
_K = 4


def _loss_kernel(*refs, nsteps, width, ncols, nstreams):
    x_refs = refs[:nstreams]
    t_ref = refs[nstreams]
    o_ref = refs[nstreams + 1]
    sum_acc = refs[nstreams + 2]
    tgt_acc = refs[nstreams + 3]
    j = pl.program_id(0)

    s = None
    tv = None
    for k in range(nstreams):
        x = x_refs[k][...]
        b, w = x.shape
        col = (j * nstreams + k) * width + jax.lax.broadcasted_iota(
            jnp.int32, (b, w), 1
        )
        e = jnp.where(col < ncols, jnp.exp(x), 0.0)
        sk = jnp.sum(e, axis=1, keepdims=True)
        tk = jnp.sum(jnp.where(col == t_ref[...], x, 0.0), axis=1, keepdims=True)
        s = sk if s is None else s + sk
        tv = tk if tv is None else tv + tk

    @pl.when(j == 0)
    def _init():
        sum_acc[...] = s
        tgt_acc[...] = tv

    @pl.when(j > 0)
    def _accum():
        sum_acc[...] += s
        tgt_acc[...] += tv

    @pl.when(j == nsteps - 1)
    def _finish():
        p = jnp.exp(tgt_acc[...]) / sum_acc[...]
        o_ref[...] = -jnp.mean(jnp.log(p + 1e-6)).reshape(1, 1)


def kernel(input, target):
    b, c = input.shape
    nsteps = pl.cdiv(c, _W * _K)
    t2 = target.astype(jnp.int32).reshape(b, 1)

    nblocks = pl.cdiv(c, _W)

    def _x_spec(k):
        return pl.BlockSpec(
            (b, _W), lambda j, _k=k: (0, jnp.minimum(j * _K + _k, nblocks - 1))
        )

    out = pl.pallas_call(
        functools.partial(
            _loss_kernel, nsteps=nsteps, width=_W, ncols=c, nstreams=_K
        ),
        grid=(nsteps,),
        in_specs=[_x_spec(k) for k in range(_K)]
        + [pl.BlockSpec((b, 1), lambda j: (0, 0))],
        out_specs=pl.BlockSpec((1, 1), lambda j: (0, 0)),
        out_shape=jax.ShapeDtypeStruct((1, 1), jnp.float32),
        scratch_shapes=[
            pltpu.VMEM((b, 1), jnp.float32),
            pltpu.VMEM((b, 1), jnp.float32),
        ],
    )(*([input] * _K), t2)
    return out[0, 0]
